# baseline (speedup 1.0000x reference)
import jax
import jax.numpy as jnp
from jax.experimental import pallas as pl

_B = 256
_MAXN = 24
_NUM_TYPES = 10
_MAXPOS = 9
_HID = 301
_LAT = 56
_HP = 304


def _mm(w, x):
    return jax.lax.dot_general(w, x, (((1,), (0,)), ((), ())),
                               preferred_element_type=jnp.float32)


def _mm_tn(xt, w):
    return jax.lax.dot_general(xt, w, (((0,), (1,)), ((), ())),
                               preferred_element_type=jnp.float32)


def _stack304(blocks):
    return jnp.concatenate(
        [jnp.pad(b, ((0, _HP - _HID), (0, 0))) for b in blocks], axis=0)


def _kern(a_ref, tflat_ref, pflat_ref, pos_ref, rcg_ref,
          wih_s_ref, whh_s_ref, b6_ref,
          wgm_h_ref, wgm_p_ref, bg_ref,
          wdf1_ref, bdf1_ref, wdf2_ref, bdf2_ref,
          wfc_h_ref, wfc_f_ref, bfc_ref,
          out_ref):
    f32 = jnp.float32
    whh_s = whh_s_ref[...]
    b6 = b6_ref[...]
    bih_r, bih_z, bih_n = b6[:, 0:1], b6[:, 1:2], b6[:, 2:3]
    bhh_r, bhh_z, bhh_n = b6[:, 3:4], b6[:, 4:5], b6[:, 5:6]
    wgm_h = wgm_h_ref[...]
    wgm_p = wgm_p_ref[...]
    bg = bg_ref[...]

    tflat = tflat_ref[...]
    pflat = pflat_ref[...]
    posq = pos_ref[...]

    iota19 = jax.lax.broadcasted_iota(
        jnp.int32, (_NUM_TYPES + _MAXPOS, _MAXN * _B), 0)
    code = jnp.where(iota19 < _NUM_TYPES, tflat, pflat + _NUM_TYPES)
    oh_all = jnp.where(code == iota19, 1.0, 0.0)
    iin_all = _mm(wih_s_ref[...], oh_all)

    iota_p = jax.lax.broadcasted_iota(jnp.int32, (_MAXPOS, _B), 0)

    grows = []
    hv = None
    for v in range(_MAXN):
        if v == 0:
            hin = jnp.zeros((_HID, _B), f32)
        else:
            terms = [jnp.where(a_ref[24 * v + u:24 * v + u + 1, :] < 0.3,
                               grows[u], 0.0)
                     for u in range(v)]
            while len(terms) > 1:
                terms = [terms[i] + terms[i + 1] if i + 1 < len(terms)
                         else terms[i] for i in range(0, len(terms), 2)]
            hin = terms[0]
        iin = iin_all[:, v * _B:(v + 1) * _B]
        if v == 0:
            hh = jnp.zeros((3 * _HP, _B), f32)
        else:
            hh = _mm(whh_s, hin)
        r = jax.nn.sigmoid(iin[0:_HID] + bih_r + hh[0:_HID] + bhh_r)
        z = jax.nn.sigmoid(iin[_HP:_HP + _HID] + bih_z
                           + hh[_HP:_HP + _HID] + bhh_z)
        n = jnp.tanh(iin[2 * _HP:2 * _HP + _HID] + bih_n
                     + r * (hh[2 * _HP:2 * _HP + _HID] + bhh_n))
        hv = (1.0 - z) * n + z * hin
        if v < _MAXN - 1:
            pv = pflat[0:1, v * _B:(v + 1) * _B]
            oh_p = jnp.where(pv == iota_p, 1.0, 0.0)
            gm = _mm(wgm_h, hv) + _mm(wgm_p, oh_p)
            gate = jax.nn.sigmoid(gm[0:_HID] + bg)
            grows.append(gate * gm[_HP:_HP + _HID])
    hg = hv

    j3 = jax.lax.broadcasted_iota(jnp.int32, (_MAXN, 3 * _MAXPOS, _B), 1)
    pj = j3 // 3
    kj = j3 - pj * 3
    niota = jax.lax.broadcasted_iota(jnp.int32, (_MAXN, 3 * _MAXPOS, _B), 0) + 1
    m27i = jnp.where(posq[:, None, :] == pj, niota, 0)
    nmax = jnp.max(m27i, axis=0)
    last = jnp.where((m27i == nmax[None, :, :]) & (m27i > 0), 1.0, 0.0)
    r3 = rcg_ref[...]
    rcg27 = (jnp.where(kj == 0, r3[:, 0:1, :], 0.0)
             + jnp.where(kj == 1, r3[:, 1:2, :], 0.0)
             + jnp.where(kj == 2, r3[:, 2:3, :], 0.0))
    df = jnp.sum(last * rcg27, axis=0)

    hdf = jnp.maximum(_mm(wdf1_ref[...], df) + bdf1_ref[...], 0.0)
    hdf = _mm(wdf2_ref[...], hdf) + bdf2_ref[...]

    out_ref[...] = (_mm_tn(hg, wfc_h_ref[...])
                    + _mm_tn(0.01 * hdf, wfc_f_ref[...]) + bfc_ref[...])


def kernel(node_types, node_pos, adj_rand, node_rcg, Wih, Whh, bih, bhh,
           Wg, bg, Wm, Wdf1, bdf1, Wdf2, bdf2, Wfc1, bfc1, Wfc2, bfc2):
    H = _HID
    a = adj_rand.transpose(2, 1, 0).reshape(_MAXN * _MAXN, _B)
    tq = node_types.T.astype(jnp.int32)
    pq = node_pos.T.astype(jnp.int32)

    args = (
        a,
        tq.reshape(1, _MAXN * _B), pq.reshape(1, _MAXN * _B), pq,
        node_rcg.transpose(1, 2, 0),
        _stack304([Wih[0:H], Wih[H:2 * H], Wih[2 * H:]]),
        _stack304([Whh[0:H], Whh[H:2 * H], Whh[2 * H:]]),
        jnp.stack([bih[0:H], bih[H:2 * H], bih[2 * H:],
                   bhh[0:H], bhh[H:2 * H], bhh[2 * H:]], axis=1),
        _stack304([Wg[:, :H], Wm[:, :H]]),
        _stack304([Wg[:, H:], Wm[:, H:]]),
        bg[:, None],
        Wdf1, bdf1[:, None], Wdf2, bdf2[:, None],
        jnp.concatenate([Wfc1[:, :H], Wfc2[:, :H]], axis=0),
        jnp.concatenate([Wfc1[:, H:], Wfc2[:, H:]], axis=0),
        jnp.concatenate([bfc1, bfc2])[None, :],
    )
    return pl.pallas_call(
        _kern,
        out_shape=jax.ShapeDtypeStruct((_B, 2 * _LAT), jnp.float32),
    )(*args)


# speedup vs baseline: 1.0391x; 1.0391x over previous
import jax
import jax.numpy as jnp
from jax.experimental import pallas as pl

_B = 256
_MAXN = 24
_NUM_TYPES = 10
_MAXPOS = 9
_HID = 301
_LAT = 56
_HP = 304


def _mm(w, x):
    return jax.lax.dot_general(w, x, (((1,), (0,)), ((), ())),
                               preferred_element_type=jnp.float32)


def _mm_tn(xt, w):
    return jax.lax.dot_general(xt, w, (((0,), (1,)), ((), ())),
                               preferred_element_type=jnp.float32)


def _stack304(blocks):
    return jnp.concatenate(
        [jnp.pad(b, ((0, _HP - _HID), (0, 0))) for b in blocks], axis=0)


def _kern(a_ref, types_ref, pos_ref, rcg_ref,
          wih_s_ref, whh_s_ref, b6_ref,
          wgm_h_ref, wgm_p_ref, bg_ref,
          wdf1_ref, bdf1_ref, wdf2_ref, bdf2_ref,
          wfc_h_ref, wfc_f_ref, bfc_ref,
          out_ref):
    f32 = jnp.float32
    wih_s = wih_s_ref[...]
    whh_s = whh_s_ref[...]
    b6 = b6_ref[...]
    bih_r, bih_z, bih_n = b6[:, 0:1], b6[:, 1:2], b6[:, 2:3]
    bhh_r, bhh_z, bhh_n = b6[:, 3:4], b6[:, 4:5], b6[:, 5:6]
    wgm_h = wgm_h_ref[...]
    wgm_p = wgm_p_ref[...]
    bg = bg_ref[...]

    types = types_ref[...]
    posq = pos_ref[...]
    iota_tp = jax.lax.broadcasted_iota(
        jnp.int32, (_NUM_TYPES + _MAXPOS, _B), 0)
    iota_p = jax.lax.broadcasted_iota(jnp.int32, (_MAXPOS, _B), 0)

    grows = []
    hv = None
    for v in range(_MAXN):
        if v == 0:
            hin = jnp.zeros((_HID, _B), f32)
        else:
            terms = [jnp.where(a_ref[24 * v + u:24 * v + u + 1, :] < 0.3,
                               grows[u], 0.0)
                     for u in range(v)]
            while len(terms) > 1:
                terms = [terms[i] + terms[i + 1] if i + 1 < len(terms)
                         else terms[i] for i in range(0, len(terms), 2)]
            hin = terms[0]
        code = jnp.where(iota_tp < _NUM_TYPES, types[v:v + 1, :],
                         posq[v:v + 1, :] + _NUM_TYPES)
        oh = jnp.where(code == iota_tp, 1.0, 0.0)
        oh_p = jnp.where(posq[v:v + 1, :] == iota_p, 1.0, 0.0)
        iin = _mm(wih_s, oh)
        if v == 0:
            hh = jnp.zeros((3 * _HP, _B), f32)
        else:
            hh = _mm(whh_s, hin)
        r = jax.nn.sigmoid(iin[0:_HID] + bih_r + hh[0:_HID] + bhh_r)
        z = jax.nn.sigmoid(iin[_HP:_HP + _HID] + bih_z
                           + hh[_HP:_HP + _HID] + bhh_z)
        n = jnp.tanh(iin[2 * _HP:2 * _HP + _HID] + bih_n
                     + r * (hh[2 * _HP:2 * _HP + _HID] + bhh_n))
        hv = (1.0 - z) * n + z * hin
        if v < _MAXN - 1:
            gm = _mm(wgm_h, hv) + _mm(wgm_p, oh_p)
            gate = jax.nn.sigmoid(gm[0:_HID] + bg)
            grows.append(gate * gm[_HP:_HP + _HID])
    hg = hv

    j3 = jax.lax.broadcasted_iota(jnp.int32, (_MAXN, 3 * _MAXPOS, _B), 1)
    pj = j3 // 3
    kj = j3 - pj * 3
    niota = jax.lax.broadcasted_iota(jnp.int32, (_MAXN, 3 * _MAXPOS, _B), 0) + 1
    m27i = jnp.where(posq[:, None, :] == pj, niota, 0)
    nmax = jnp.max(m27i, axis=0)
    last = jnp.where((m27i == nmax[None, :, :]) & (m27i > 0), 1.0, 0.0)
    r3 = rcg_ref[...]
    rcg27 = (jnp.where(kj == 0, r3[:, 0:1, :], 0.0)
             + jnp.where(kj == 1, r3[:, 1:2, :], 0.0)
             + jnp.where(kj == 2, r3[:, 2:3, :], 0.0))
    df = jnp.sum(last * rcg27, axis=0)

    hdf = jnp.maximum(_mm(wdf1_ref[...], df) + bdf1_ref[...], 0.0)
    hdf = _mm(wdf2_ref[...], hdf) + bdf2_ref[...]

    out_ref[...] = (_mm_tn(hg, wfc_h_ref[...])
                    + _mm_tn(0.01 * hdf, wfc_f_ref[...]) + bfc_ref[...])


def kernel(node_types, node_pos, adj_rand, node_rcg, Wih, Whh, bih, bhh,
           Wg, bg, Wm, Wdf1, bdf1, Wdf2, bdf2, Wfc1, bfc1, Wfc2, bfc2):
    H = _HID
    a = adj_rand.transpose(2, 1, 0).reshape(_MAXN * _MAXN, _B)

    args = (
        a, node_types.T.astype(jnp.int32), node_pos.T.astype(jnp.int32),
        node_rcg.transpose(1, 2, 0),
        _stack304([Wih[0:H], Wih[H:2 * H], Wih[2 * H:]]),
        _stack304([Whh[0:H], Whh[H:2 * H], Whh[2 * H:]]),
        jnp.stack([bih[0:H], bih[H:2 * H], bih[2 * H:],
                   bhh[0:H], bhh[H:2 * H], bhh[2 * H:]], axis=1),
        _stack304([Wg[:, :H], Wm[:, :H]]),
        _stack304([Wg[:, H:], Wm[:, H:]]),
        bg[:, None],
        Wdf1, bdf1[:, None], Wdf2, bdf2[:, None],
        jnp.concatenate([Wfc1[:, :H], Wfc2[:, :H]], axis=0),
        jnp.concatenate([Wfc1[:, H:], Wfc2[:, H:]], axis=0),
        jnp.concatenate([bfc1, bfc2])[None, :],
    )
    return pl.pallas_call(
        _kern,
        out_shape=jax.ShapeDtypeStruct((_B, 2 * _LAT), jnp.float32),
    )(*args)
